# Initial kernel scaffold; baseline (speedup 1.0000x reference)
import jax
import jax.numpy as jnp
from jax.experimental import pallas as pl


def kernel(input_ids, token_type_ids, history_encoding, turn_encoding, scenario_encoding, word_emb, pos_emb, type_emb, hist_emb, turn_emb, gamma, beta):
    raise NotImplementedError("write your pallas kernel here")


# speedup vs baseline: 1.2146x; 1.2146x over previous
import jax
import jax.numpy as jnp
from jax.experimental import pallas as pl


def _ln_body(x_ref, g_ref, b_ref, o_ref):
    x = x_ref[...]
    mu = jnp.mean(x, axis=-1, keepdims=True)
    var = jnp.mean((x - mu) ** 2, axis=-1, keepdims=True)
    normed = (x - mu) * jax.lax.rsqrt(var + 1e-12)
    o_ref[...] = g_ref[...] * normed + b_ref[...]


def kernel(input_ids, token_type_ids, history_encoding, turn_encoding, scenario_encoding,
           word_emb, pos_emb, type_emb, hist_emb, turn_emb, gamma, beta):
    B, S = input_ids.shape
    H = word_emb.shape[1]
    emb = (jnp.take(word_emb, input_ids, axis=0)
           + pos_emb[:S][None, :, :]
           + jnp.take(type_emb, token_type_ids, axis=0)
           + jnp.take(hist_emb, history_encoding, axis=0)
           + jnp.take(hist_emb, scenario_encoding, axis=0)
           + jnp.take(turn_emb, turn_encoding, axis=0))
    BLK = 64
    out = pl.pallas_call(
        _ln_body,
        grid=(B // BLK,),
        in_specs=[
            pl.BlockSpec((BLK, S, H), lambda i: (i, 0, 0)),
            pl.BlockSpec((H,), lambda i: (0,)),
            pl.BlockSpec((H,), lambda i: (0,)),
        ],
        out_specs=pl.BlockSpec((BLK, S, H), lambda i: (i, 0, 0)),
        out_shape=jax.ShapeDtypeStruct((B, S, H), jnp.float32),
    )(emb, gamma, beta)
    return out


# speedup vs baseline: 7.9570x; 6.5511x over previous
import functools

import jax
import jax.numpy as jnp
from jax import lax
from jax.experimental import pallas as pl
from jax.experimental.pallas import tpu as pltpu
from jax.experimental.pallas import tpu_sc as plsc

NC, NS, L = 2, 16, 16
NW = NC * NS
NJ = None


_GATHER_DNUMS = lax.GatherDimensionNumbers(
    offset_dims=(), collapsed_slice_dims=(0,), start_index_map=(0,))


def _permute(x, idx):
    return lax.gather(x, idx[:, None], _GATHER_DNUMS, (1,),
                      mode=lax.GatherScatterMode.PROMISE_IN_BOUNDS)


def _xlane_sum(x):
    for k in (8, 4, 2, 1):
        idx = lax.iota(jnp.int32, L) ^ k
        x = x + _permute(x, idx)
    return x


def _rsqrt_nr(x):
    i = lax.bitcast_convert_type(x, jnp.int32)
    i = jnp.int32(0x5F3759DF) - (i >> 1)
    y = lax.bitcast_convert_type(i, jnp.float32)
    half = x * 0.5
    for _ in range(3):
        y = y * (1.5 - half * y * y)
    return y


def _make_sc_kernel(B, S, H, VOCAB, MAX_POS):
    rows_per_w = B // NW
    nj = H // L
    SP = ((S + L - 1) // L) * L
    mesh = plsc.VectorSubcoreMesh(
        core_axis_name="c", subcore_axis_name="s",
        num_cores=NC, num_subcores=NS)

    @functools.partial(
        pl.kernel,
        out_type=jax.ShapeDtypeStruct((B, S, H), jnp.float32),
        mesh=mesh,
        scratch_types=[
            pltpu.VMEM((S, H), jnp.float32),
            pltpu.VMEM((256, H), jnp.float32),
            pltpu.VMEM((2, H), jnp.float32),
            pltpu.VMEM((4, H), jnp.float32),
            pltpu.VMEM((8, H), jnp.float32),
            pltpu.VMEM((H,), jnp.float32),
            pltpu.VMEM((H,), jnp.float32),
            pltpu.VMEM((S, H), jnp.float32),
            pltpu.VMEM((SP,), jnp.int32),
            pltpu.VMEM((SP,), jnp.int32),
            pltpu.VMEM((SP,), jnp.int32),
            pltpu.VMEM((SP,), jnp.int32),
            pltpu.VMEM((SP,), jnp.int32),
            pltpu.VMEM((SP,), jnp.int32),
            pltpu.SemaphoreType.DMA,
        ],
    )
    def k(ids_hbm, tt_hbm, hi_hbm, tu_hbm, sc_hbm,
          word_hbm, pos_hbm, type_hbm, hist_hbm, turn_hbm, g_hbm, b_hbm,
          out_hbm,
          pos_v, comb_v, t2_v, h4_v, t8_v, g_v, b_v,
          rows_v, wid_v, tt_v, hi_v, tu_v, sc_v, cid_v, sem):
        wid = lax.axis_index("s") * NC + lax.axis_index("c")

        pltpu.sync_copy(pos_hbm.at[pl.ds(0, S)], pos_v)
        pltpu.sync_copy(type_hbm, t2_v)
        pltpu.sync_copy(hist_hbm, h4_v)
        pltpu.sync_copy(turn_hbm, t8_v)
        pltpu.sync_copy(g_hbm, g_v)
        pltpu.sync_copy(b_hbm, b_v)

        def comb_body(cid, _):
            tt = cid >> 7
            h = (cid >> 5) & 3
            sc = (cid >> 3) & 3
            t = cid & 7
            for j in range(nj):
                d = pl.ds(j * L, L)
                comb_v[cid, d] = (t2_v[tt, d] + h4_v[h, d]
                                  + h4_v[sc, d] + t8_v[t, d])
            return ()
        lax.fori_loop(0, 256, comb_body, (), unroll=False)

        gs = [g_v[pl.ds(j * L, L)] for j in range(nj)]
        bs = [b_v[pl.ds(j * L, L)] for j in range(nj)]

        zero = jnp.zeros((L,), jnp.int32)
        for buf in (tt_v, hi_v, tu_v, sc_v):
            buf[pl.ds(SP - L, L)] = zero

        def row_body(r, _):
            b = wid * rows_per_w + r
            tok0 = b * S
            pltpu.sync_copy(ids_hbm.at[pl.ds(tok0, S)], wid_v.at[pl.ds(0, S)])
            pltpu.sync_copy(tt_hbm.at[pl.ds(tok0, S)], tt_v.at[pl.ds(0, S)])
            pltpu.sync_copy(hi_hbm.at[pl.ds(tok0, S)], hi_v.at[pl.ds(0, S)])
            pltpu.sync_copy(tu_hbm.at[pl.ds(tok0, S)], tu_v.at[pl.ds(0, S)])
            pltpu.sync_copy(sc_hbm.at[pl.ds(tok0, S)], sc_v.at[pl.ds(0, S)])
            cp0 = pltpu.async_copy(
                word_hbm.at[wid_v.at[pl.ds(0, 128)]],
                rows_v.at[pl.ds(0, 128)], sem)
            cp1 = pltpu.async_copy(
                word_hbm.at[wid_v.at[pl.ds(128, S - 128)]],
                rows_v.at[pl.ds(128, S - 128)], sem)
            for kk in range(SP // L):
                d = pl.ds(kk * L, L)
                cid_v[d] = ((tt_v[d] * 4 + hi_v[d]) * 4 + sc_v[d]) * 8 + tu_v[d]
            cp0.wait()
            cp1.wait()

            def do_token(s, cid):
                acc = []
                for j in range(nj):
                    d = pl.ds(j * L, L)
                    acc.append(rows_v[s, d] + comb_v[cid, d] + pos_v[s, d])
                tot = acc[0]
                sq = acc[0] * acc[0]
                for j in range(1, nj):
                    tot = tot + acc[j]
                    sq = sq + acc[j] * acc[j]
                mu = _xlane_sum(tot) * (1.0 / H)
                msq = _xlane_sum(sq) * (1.0 / H)
                var = msq - mu * mu
                rinv = _rsqrt_nr(var + 1e-12)
                for j in range(nj):
                    d = pl.ds(j * L, L)
                    rows_v[s, d] = (acc[j] - mu) * (rinv * gs[j]) + bs[j]

            def grp_body(g, _):
                s0 = g * L
                cid_vec = cid_v[pl.ds(s0, L)]
                for t in range(L):
                    do_token(s0 + t, cid_vec[t])
                return ()
            lax.fori_loop(0, S // L, grp_body, (), unroll=False)
            if S % L:
                cid_vec = cid_v[pl.ds(SP - L, L)]
                for t in range(S % L):
                    do_token(SP - L + t, cid_vec[t])
            pltpu.sync_copy(rows_v, out_hbm.at[b])
            return ()
        lax.fori_loop(0, rows_per_w, row_body, (), unroll=False)

    return k


def kernel(input_ids, token_type_ids, history_encoding, turn_encoding, scenario_encoding,
           word_emb, pos_emb, type_emb, hist_emb, turn_emb, gamma, beta):
    B, S = input_ids.shape
    VOCAB, H = word_emb.shape
    k = _make_sc_kernel(B, S, H, VOCAB, pos_emb.shape[0])
    flat = lambda a: a.astype(jnp.int32).reshape(-1)
    return k(flat(input_ids), flat(token_type_ids), flat(history_encoding),
             flat(turn_encoding), flat(scenario_encoding),
             word_emb, pos_emb, type_emb, hist_emb, turn_emb, gamma, beta)
